# baseline (speedup 1.0000x reference)
import jax
import jax.numpy as jnp
from jax import lax
from jax.experimental import pallas as pl
from jax.experimental.pallas import tpu as pltpu
from jax.experimental.pallas import tpu_sc as plsc

VOCAB = 1000000
D = 32
PW = 16
NCLS = 2
B = 4096
S = 200
NC = 2
NS = 16
NW = NC * NS
RPW = B // NW
C0 = 128
C1 = S - C0
PC = 8192
G = 128 // PW
GC = PC // G
NBLK = (VOCAB + PC - 1) // PC
VPAD = NBLK * PC


def _proj_body(t_ref, wb_ref, out_ref):
    tbig = t_ref[...].reshape(D * G, GC)
    out_ref[...] = lax.dot_general(
        tbig, wb_ref[...], (((0,), (0,)), ((), ())),
        preferred_element_type=jnp.float32)


def _project(tableT, wb):
    nlines = VPAD * PW // 128
    return pl.pallas_call(
        _proj_body,
        grid=(NBLK,),
        in_specs=[
            pl.BlockSpec((D, PC), lambda i: (0, i)),
            pl.BlockSpec((D * G, 128), lambda i: (0, 0)),
        ],
        out_specs=pl.BlockSpec((GC, 128), lambda i: (i, 0)),
        out_shape=jax.ShapeDtypeStruct((nlines, 128), jnp.float32),
    )(tableT, wb)


def _sc_body(ids_hbm, p_hbm, sums_hbm, ids_v, idx_v, sums_v,
             buf0, buf1, buf2, buf3, sem0, sem1, sem2, sem3):
    wid = lax.axis_index("s") * NC + lax.axis_index("c")
    base = wid * RPW

    pltpu.sync_copy(ids_hbm.at[pl.ds(base, RPW)], ids_v)

    def perm_row(i):
        for j in list(range(0, S - PW, PW)) + [S - PW]:
            v = ids_v[i, pl.ds(j, PW)]
            s = ((v & -8192) + ((v & 1023) << 3) +
                 ((v >> 10) & 7))
            idx_v[i, pl.ds(j, PW)] = s

    def copies(i, buf, sem):
        cp0 = pltpu.make_async_copy(
            p_hbm.at[idx_v.at[i, pl.ds(0, C0)]], buf.at[pl.ds(0, C0)], sem)
        cp1 = pltpu.make_async_copy(
            p_hbm.at[idx_v.at[i, pl.ds(C0, C1)]], buf.at[pl.ds(C0, C1)],
            sem)
        return cp0, cp1

    def issue(i, buf, sem):
        cp0, cp1 = copies(i, buf, sem)
        cp0.start()
        cp1.start()

    def drain(i, buf, sem):
        cp0, cp1 = copies(i, buf, sem)
        cp0.wait()
        cp1.wait()

    NBUF = 4

    def process(i, buf, sem):
        drain(i, buf, sem)
        a = [buf[j, 0:PW] for j in range(4)]
        for j in range(4, S, 4):
            for k in range(4):
                a[k] = a[k] + buf[j + k, 0:PW]
        acc = (a[0] + a[1]) + (a[2] + a[3])

        @pl.when(i + NBUF < RPW)
        def _():
            perm_row(i + NBUF)
            issue(i + NBUF, buf, sem)

        sums_v[i, 0:PW] = acc

    bufs = [buf0, buf1, buf2, buf3]
    sems = [sem0, sem1, sem2, sem3]
    for k in range(NBUF):
        perm_row(k)
        issue(k, bufs[k], sems[k])

    def body(g, carry):
        for k in range(NBUF):
            process(NBUF * g + k, bufs[k], sems[k])
        return carry

    lax.fori_loop(0, RPW // NBUF, body, 0)

    pltpu.sync_copy(sums_v, sums_hbm.at[pl.ds(base, RPW)])


def _make_sc_entry():
    mesh = plsc.VectorSubcoreMesh(core_axis_name="c", subcore_axis_name="s")
    return pl.kernel(
        _sc_body,
        mesh=mesh,
        compiler_params=pltpu.CompilerParams(use_tc_tiling_on_sc=False),
        out_type=jax.ShapeDtypeStruct((B, PW), jnp.float32),
        scratch_types=[
            pltpu.VMEM((RPW, S), jnp.int32),
            pltpu.VMEM((RPW, S), jnp.int32),
            pltpu.VMEM((RPW, PW), jnp.float32),
            pltpu.VMEM((S, PW), jnp.float32),
            pltpu.VMEM((S, PW), jnp.float32),
            pltpu.VMEM((S, PW), jnp.float32),
            pltpu.VMEM((S, PW), jnp.float32),
            pltpu.SemaphoreType.DMA,
            pltpu.SemaphoreType.DMA,
            pltpu.SemaphoreType.DMA,
            pltpu.SemaphoreType.DMA,
        ],
    )


_sc_entry = _make_sc_entry()


def _tc_body(sums_ref, len_ref, b_ref, out_ref):
    inv = 1.0 / jnp.maximum(len_ref[...], 1).astype(jnp.float32)
    out_ref[...] = sums_ref[:, 0:NCLS] * inv + b_ref[...]


def _classify(sums, lengths2d, b2d):
    return pl.pallas_call(
        _tc_body,
        out_shape=jax.ShapeDtypeStruct((B, NCLS), jnp.float32),
    )(sums, lengths2d, b2d)


def kernel(input_ids, lengths, emb_table, W, b):
    w16 = jnp.zeros((PW, D), jnp.float32).at[:NCLS].set(W)
    lane = jnp.arange(128)
    m = w16.T[:, lane & 15]
    gmask = ((lane >> 4)[None, None, :] ==
             jnp.arange(G)[None, :, None]).astype(jnp.float32)
    wb = (m[:, None, :] * gmask).reshape(D * G, 128)
    p2d = _project(emb_table.T, wb)
    sums = _sc_entry(input_ids.astype(jnp.int32), p2d.reshape(VPAD, PW))
    return _classify(sums, lengths.astype(jnp.int32).reshape(B, 1),
                     b.reshape(1, NCLS))


# speedup vs baseline: 1.1971x; 1.1971x over previous
import jax
import jax.numpy as jnp
from jax import lax
from jax.experimental import pallas as pl
from jax.experimental.pallas import tpu as pltpu
from jax.experimental.pallas import tpu_sc as plsc

VOCAB = 1000000
D = 32
PW = 16
NCLS = 2
B = 4096
S = 200
NC = 2
NS = 16
NW = NC * NS
RPW = B // NW
C0 = 128
C1 = S - C0
PC = 8192
G = 128 // PW
GC = PC // G
NBLK = (VOCAB + PC - 1) // PC
VPAD = NBLK * PC


def _proj_body(t_ref, wb_ref, out_ref):
    tbig = t_ref[...].reshape(D * G, GC)
    out_ref[...] = lax.dot_general(
        tbig, wb_ref[...], (((0,), (0,)), ((), ())),
        preferred_element_type=jnp.float32)


def _project(tableT, wb):
    nlines = VPAD * PW // 128
    return pl.pallas_call(
        _proj_body,
        grid=(NBLK,),
        in_specs=[
            pl.BlockSpec((D, PC), lambda i: (0, i)),
            pl.BlockSpec((D * G, 128), lambda i: (0, 0)),
        ],
        out_specs=pl.BlockSpec((GC, 128), lambda i: (i, 0)),
        out_shape=jax.ShapeDtypeStruct((nlines, 128), jnp.float32),
    )(tableT, wb)


def _sc_body(ids_hbm, p_hbm, sums_hbm, ids_v, idx_v, sums_v,
             buf0, buf1, buf2, buf3, sem0, sem1, sem2, sem3):
    wid = lax.axis_index("s") * NC + lax.axis_index("c")
    base = wid * RPW

    pltpu.sync_copy(ids_hbm.at[pl.ds(base, RPW)], ids_v)

    def perm_row(i):
        for j in list(range(0, S - PW, PW)) + [S - PW]:
            v = ids_v[i, pl.ds(j, PW)]
            s = ((v & -8192) + ((v & 1023) << 3) +
                 ((v >> 10) & 7))
            idx_v[i, pl.ds(j, PW)] = s

    def copies(i, buf, sem):
        cp0 = pltpu.make_async_copy(
            p_hbm.at[idx_v.at[i, pl.ds(0, C0)]], buf.at[pl.ds(0, C0)], sem)
        cp1 = pltpu.make_async_copy(
            p_hbm.at[idx_v.at[i, pl.ds(C0, C1)]], buf.at[pl.ds(C0, C1)],
            sem)
        return cp0, cp1

    def issue(i, buf, sem):
        cp0, cp1 = copies(i, buf, sem)
        cp0.start()
        cp1.start()

    def drain(i, buf, sem):
        cp0, cp1 = copies(i, buf, sem)
        cp0.wait()
        cp1.wait()

    NBUF = 4

    def process(i, buf, sem):
        drain(i, buf, sem)
        a = [buf[j, 0:PW] for j in range(4)]
        for j in range(4, S, 4):
            for k in range(4):
                a[k] = a[k] + buf[j + k, 0:PW]
        acc = (a[0] + a[1]) + (a[2] + a[3])

        @pl.when(i + NBUF < RPW)
        def _():
            issue(i + NBUF, buf, sem)

        sums_v[i, 0:PW] = acc

    def perm_all(i, carry):
        perm_row(i)
        return carry

    lax.fori_loop(0, RPW, perm_all, 0)

    bufs = [buf0, buf1, buf2, buf3]
    sems = [sem0, sem1, sem2, sem3]
    for k in range(NBUF):
        issue(k, bufs[k], sems[k])

    def body(g, carry):
        for k in range(NBUF):
            process(NBUF * g + k, bufs[k], sems[k])
        return carry

    lax.fori_loop(0, RPW // NBUF, body, 0)

    pltpu.sync_copy(sums_v, sums_hbm.at[pl.ds(base, RPW)])


def _make_sc_entry():
    mesh = plsc.VectorSubcoreMesh(core_axis_name="c", subcore_axis_name="s")
    return pl.kernel(
        _sc_body,
        mesh=mesh,
        compiler_params=pltpu.CompilerParams(use_tc_tiling_on_sc=False),
        out_type=jax.ShapeDtypeStruct((B, PW), jnp.float32),
        scratch_types=[
            pltpu.VMEM((RPW, S), jnp.int32),
            pltpu.VMEM((RPW, S), jnp.int32),
            pltpu.VMEM((RPW, PW), jnp.float32),
            pltpu.VMEM((S, PW), jnp.float32),
            pltpu.VMEM((S, PW), jnp.float32),
            pltpu.VMEM((S, PW), jnp.float32),
            pltpu.VMEM((S, PW), jnp.float32),
            pltpu.SemaphoreType.DMA,
            pltpu.SemaphoreType.DMA,
            pltpu.SemaphoreType.DMA,
            pltpu.SemaphoreType.DMA,
        ],
    )


_sc_entry = _make_sc_entry()


def _tc_body(sums_ref, len_ref, b_ref, out_ref):
    inv = 1.0 / jnp.maximum(len_ref[...], 1).astype(jnp.float32)
    out_ref[...] = sums_ref[:, 0:NCLS] * inv + b_ref[...]


def _classify(sums, lengths2d, b2d):
    return pl.pallas_call(
        _tc_body,
        out_shape=jax.ShapeDtypeStruct((B, NCLS), jnp.float32),
    )(sums, lengths2d, b2d)


def kernel(input_ids, lengths, emb_table, W, b):
    w16 = jnp.zeros((PW, D), jnp.float32).at[:NCLS].set(W)
    lane = jnp.arange(128)
    m = w16.T[:, lane & 15]
    gmask = ((lane >> 4)[None, None, :] ==
             jnp.arange(G)[None, :, None]).astype(jnp.float32)
    wb = (m[:, None, :] * gmask).reshape(D * G, 128)
    p2d = _project(emb_table.T, wb)
    sums = _sc_entry(input_ids.astype(jnp.int32), p2d.reshape(VPAD, PW))
    return _classify(sums, lengths.astype(jnp.int32).reshape(B, 1),
                     b.reshape(1, NCLS))


# speedup vs baseline: 1.6208x; 1.3540x over previous
import jax
import jax.numpy as jnp
from jax import lax
from jax.experimental import pallas as pl
from jax.experimental.pallas import tpu as pltpu
from jax.experimental.pallas import tpu_sc as plsc

VOCAB = 1000000
D = 32
PW = 16
NCLS = 2
B = 4096
S = 200
NC = 2
NS = 16
NW = NC * NS
RPW = B // NW
C0 = 128
C1 = S - C0
PC = 32768
G = 128 // PW
GC = PC // G
LOG2GC = GC.bit_length() - 1
NBLK = (VOCAB + PC - 1) // PC
VPAD = NBLK * PC


def _proj_body(t_ref, wb_ref, out_ref):
    tbig = t_ref[...].reshape(D * G, GC)
    out_ref[...] = lax.dot_general(
        tbig, wb_ref[...], (((0,), (0,)), ((), ())),
        preferred_element_type=jnp.float32)


def _project(tableT, wb):
    nlines = VPAD * PW // 128
    return pl.pallas_call(
        _proj_body,
        grid=(NBLK,),
        in_specs=[
            pl.BlockSpec((D, PC), lambda i: (0, i)),
            pl.BlockSpec((D * G, 128), lambda i: (0, 0)),
        ],
        out_specs=pl.BlockSpec((GC, 128), lambda i: (i, 0)),
        out_shape=jax.ShapeDtypeStruct((nlines, 128), jnp.float32),
    )(tableT, wb)


def _sc_body(ids_hbm, p_hbm, sums_hbm, ids_v, idx_v, sums_v,
             buf0, buf1, buf2, buf3, sem0, sem1, sem2, sem3):
    wid = lax.axis_index("s") * NC + lax.axis_index("c")
    base = wid * RPW

    pltpu.sync_copy(ids_hbm.at[pl.ds(base, RPW)], ids_v)

    def perm_row(i):
        for j in list(range(0, S - PW, PW)) + [S - PW]:
            v = ids_v[i, pl.ds(j, PW)]
            s = ((v & -PC) + ((v & (GC - 1)) << 3) +
                 ((v >> LOG2GC) & (G - 1)))
            idx_v[i, pl.ds(j, PW)] = s

    def copies(i, buf, sem):
        cp0 = pltpu.make_async_copy(
            p_hbm.at[idx_v.at[i, pl.ds(0, C0)]], buf.at[pl.ds(0, C0)], sem)
        cp1 = pltpu.make_async_copy(
            p_hbm.at[idx_v.at[i, pl.ds(C0, C1)]], buf.at[pl.ds(C0, C1)],
            sem)
        return cp0, cp1

    def issue(i, buf, sem):
        cp0, cp1 = copies(i, buf, sem)
        cp0.start()
        cp1.start()

    def drain(i, buf, sem):
        cp0, cp1 = copies(i, buf, sem)
        cp0.wait()
        cp1.wait()

    NBUF = 4

    def process(i, buf, sem):
        drain(i, buf, sem)
        a = [buf[j, 0:PW] for j in range(4)]
        for j in range(4, S, 4):
            for k in range(4):
                a[k] = a[k] + buf[j + k, 0:PW]
        acc = (a[0] + a[1]) + (a[2] + a[3])

        @pl.when(i + NBUF < RPW)
        def _():
            issue(i + NBUF, buf, sem)

        sums_v[i, 0:PW] = acc

    def perm_all(i, carry):
        perm_row(i)
        return carry

    lax.fori_loop(0, RPW, perm_all, 0)

    bufs = [buf0, buf1, buf2, buf3]
    sems = [sem0, sem1, sem2, sem3]
    for k in range(NBUF):
        issue(k, bufs[k], sems[k])

    def body(g, carry):
        for k in range(NBUF):
            process(NBUF * g + k, bufs[k], sems[k])
        return carry

    lax.fori_loop(0, RPW // NBUF, body, 0)

    pltpu.sync_copy(sums_v, sums_hbm.at[pl.ds(base, RPW)])


def _make_sc_entry():
    mesh = plsc.VectorSubcoreMesh(core_axis_name="c", subcore_axis_name="s")
    return pl.kernel(
        _sc_body,
        mesh=mesh,
        compiler_params=pltpu.CompilerParams(use_tc_tiling_on_sc=False),
        out_type=jax.ShapeDtypeStruct((B, PW), jnp.float32),
        scratch_types=[
            pltpu.VMEM((RPW, S), jnp.int32),
            pltpu.VMEM((RPW, S), jnp.int32),
            pltpu.VMEM((RPW, PW), jnp.float32),
            pltpu.VMEM((S, PW), jnp.float32),
            pltpu.VMEM((S, PW), jnp.float32),
            pltpu.VMEM((S, PW), jnp.float32),
            pltpu.VMEM((S, PW), jnp.float32),
            pltpu.SemaphoreType.DMA,
            pltpu.SemaphoreType.DMA,
            pltpu.SemaphoreType.DMA,
            pltpu.SemaphoreType.DMA,
        ],
    )


_sc_entry = _make_sc_entry()


def _tc_body(sums_ref, len_ref, b_ref, out_ref):
    inv = 1.0 / jnp.maximum(len_ref[...], 1).astype(jnp.float32)
    out_ref[...] = sums_ref[:, 0:NCLS] * inv + b_ref[...]


def _classify(sums, lengths2d, b2d):
    return pl.pallas_call(
        _tc_body,
        out_shape=jax.ShapeDtypeStruct((B, NCLS), jnp.float32),
    )(sums, lengths2d, b2d)


def kernel(input_ids, lengths, emb_table, W, b):
    w16 = jnp.zeros((PW, D), jnp.float32).at[:NCLS].set(W)
    lane = jnp.arange(128)
    m = w16.T[:, lane & 15]
    gmask = ((lane >> 4)[None, None, :] ==
             jnp.arange(G)[None, :, None]).astype(jnp.float32)
    wb = (m[:, None, :] * gmask).reshape(D * G, 128)
    p2d = _project(emb_table.T, wb)
    sums = _sc_entry(input_ids.astype(jnp.int32), p2d.reshape(VPAD, PW))
    return _classify(sums, lengths.astype(jnp.int32).reshape(B, 1),
                     b.reshape(1, NCLS))


# speedup vs baseline: 1.6986x; 1.0480x over previous
import jax
import jax.numpy as jnp
from jax import lax
from jax.experimental import pallas as pl
from jax.experimental.pallas import tpu as pltpu
from jax.experimental.pallas import tpu_sc as plsc

VOCAB = 1000000
D = 32
PW = 16
NCLS = 2
B = 4096
S = 200
NC = 2
NS = 16
NW = NC * NS
RPW = B // NW
C0 = 128
C1 = S - C0
PC = 65536
G = 128 // PW
GC = PC // G
LOG2GC = GC.bit_length() - 1
NBLK = (VOCAB + PC - 1) // PC
VPAD = NBLK * PC


def _proj_body(t_ref, wb_ref, out_ref):
    tbig = t_ref[...].reshape(D * G, GC)
    out_ref[...] = lax.dot_general(
        tbig, wb_ref[...], (((0,), (0,)), ((), ())),
        preferred_element_type=jnp.float32)


def _project(tableT, wb):
    nlines = VPAD * PW // 128
    return pl.pallas_call(
        _proj_body,
        grid=(NBLK,),
        in_specs=[
            pl.BlockSpec((D, PC), lambda i: (0, i)),
            pl.BlockSpec((D * G, 128), lambda i: (0, 0)),
        ],
        out_specs=pl.BlockSpec((GC, 128), lambda i: (i, 0)),
        out_shape=jax.ShapeDtypeStruct((nlines, 128), jnp.float32),
    )(tableT, wb)


def _sc_body(ids_hbm, p_hbm, sums_hbm, ids_v, idx_v, sums_v,
             buf0, buf1, buf2, buf3, sem0, sem1, sem2, sem3):
    wid = lax.axis_index("s") * NC + lax.axis_index("c")
    base = wid * RPW

    pltpu.sync_copy(ids_hbm.at[pl.ds(base, RPW)], ids_v)

    def perm_row(i):
        for j in list(range(0, S - PW, PW)) + [S - PW]:
            v = ids_v[i, pl.ds(j, PW)]
            s = ((v & -PC) + ((v & (GC - 1)) << 3) +
                 ((v >> LOG2GC) & (G - 1)))
            idx_v[i, pl.ds(j, PW)] = s

    def copies(i, buf, sem):
        cp0 = pltpu.make_async_copy(
            p_hbm.at[idx_v.at[i, pl.ds(0, C0)]], buf.at[pl.ds(0, C0)], sem)
        cp1 = pltpu.make_async_copy(
            p_hbm.at[idx_v.at[i, pl.ds(C0, C1)]], buf.at[pl.ds(C0, C1)],
            sem)
        return cp0, cp1

    def issue(i, buf, sem):
        cp0, cp1 = copies(i, buf, sem)
        cp0.start()
        cp1.start()

    def drain(i, buf, sem):
        cp0, cp1 = copies(i, buf, sem)
        cp0.wait()
        cp1.wait()

    NBUF = 4

    def process(i, buf, sem):
        drain(i, buf, sem)
        a = [buf[j, 0:PW] for j in range(4)]
        for j in range(4, S, 4):
            for k in range(4):
                a[k] = a[k] + buf[j + k, 0:PW]
        acc = (a[0] + a[1]) + (a[2] + a[3])

        @pl.when(i + NBUF < RPW)
        def _():
            issue(i + NBUF, buf, sem)

        sums_v[i, 0:PW] = acc

    def perm_all(i, carry):
        perm_row(i)
        return carry

    lax.fori_loop(0, RPW, perm_all, 0)

    bufs = [buf0, buf1, buf2, buf3]
    sems = [sem0, sem1, sem2, sem3]
    for k in range(NBUF):
        issue(k, bufs[k], sems[k])

    def body(g, carry):
        for k in range(NBUF):
            process(NBUF * g + k, bufs[k], sems[k])
        return carry

    lax.fori_loop(0, RPW // NBUF, body, 0)

    pltpu.sync_copy(sums_v, sums_hbm.at[pl.ds(base, RPW)])


def _make_sc_entry():
    mesh = plsc.VectorSubcoreMesh(core_axis_name="c", subcore_axis_name="s")
    return pl.kernel(
        _sc_body,
        mesh=mesh,
        compiler_params=pltpu.CompilerParams(use_tc_tiling_on_sc=False),
        out_type=jax.ShapeDtypeStruct((B, PW), jnp.float32),
        scratch_types=[
            pltpu.VMEM((RPW, S), jnp.int32),
            pltpu.VMEM((RPW, S), jnp.int32),
            pltpu.VMEM((RPW, PW), jnp.float32),
            pltpu.VMEM((S, PW), jnp.float32),
            pltpu.VMEM((S, PW), jnp.float32),
            pltpu.VMEM((S, PW), jnp.float32),
            pltpu.VMEM((S, PW), jnp.float32),
            pltpu.SemaphoreType.DMA,
            pltpu.SemaphoreType.DMA,
            pltpu.SemaphoreType.DMA,
            pltpu.SemaphoreType.DMA,
        ],
    )


_sc_entry = _make_sc_entry()


def _tc_body(sums_ref, len_ref, b_ref, out_ref):
    inv = 1.0 / jnp.maximum(len_ref[...], 1).astype(jnp.float32)
    out_ref[...] = sums_ref[:, 0:NCLS] * inv + b_ref[...]


def _classify(sums, lengths2d, b2d):
    return pl.pallas_call(
        _tc_body,
        out_shape=jax.ShapeDtypeStruct((B, NCLS), jnp.float32),
    )(sums, lengths2d, b2d)


def kernel(input_ids, lengths, emb_table, W, b):
    w16 = jnp.zeros((PW, D), jnp.float32).at[:NCLS].set(W)
    lane = jnp.arange(128)
    m = w16.T[:, lane & 15]
    gmask = ((lane >> 4)[None, None, :] ==
             jnp.arange(G)[None, :, None]).astype(jnp.float32)
    wb = (m[:, None, :] * gmask).reshape(D * G, 128)
    p2d = _project(emb_table.T, wb)
    sums = _sc_entry(input_ids.astype(jnp.int32), p2d.reshape(VPAD, PW))
    return _classify(sums, lengths.astype(jnp.int32).reshape(B, 1),
                     b.reshape(1, NCLS))
